# baseline (speedup 1.0000x reference)
import functools

import numpy as np
import jax
import jax.numpy as jnp
from jax import lax
from jax.experimental import pallas as pl
from jax.experimental.pallas import tpu as pltpu
from jax.experimental.pallas import tpu_sc as plsc

D_MODEL = 768
VOCAB = 100000
BATCH = 4
SEQ = 8192

SCALE = float(np.sqrt(np.float32(D_MODEL)))

NUM_CORES = 2
NUM_SUBCORES = 16
NUM_WORKERS = NUM_CORES * NUM_SUBCORES
POS_PER_WORKER = SEQ // NUM_WORKERS
CHUNK = 32
N_CHUNKS = POS_PER_WORKER // CHUNK
LANES = 16
D_GROUPS = D_MODEL // LANES


def _sinusoidal_pe(length, d_model):
    pos = np.arange(length)[:, None].astype(np.float32)
    i = np.arange(d_model)[None, :].astype(np.float32)
    angle_rates = 1.0 / np.power(10000.0, (2.0 * (i // 2)) / np.float32(d_model))
    angles = pos * angle_rates
    pe = np.zeros((length, d_model), dtype=np.float32)
    pe[:, 0::2] = np.sin(angles[:, 0::2])
    pe[:, 1::2] = np.cos(angles[:, 1::2])
    return pe


_PE = _sinusoidal_pe(SEQ, D_MODEL)

_MESH = plsc.VectorSubcoreMesh(core_axis_name="c", subcore_axis_name="s")


@functools.partial(
    pl.kernel,
    out_type=jax.ShapeDtypeStruct((BATCH, SEQ, D_MODEL), jnp.float32),
    mesh=_MESH,
    scratch_types=[
        pltpu.VMEM((CHUNK,), jnp.int32),
        pltpu.VMEM((CHUNK, D_MODEL), jnp.float32),
        pltpu.VMEM((CHUNK, D_MODEL), jnp.float32),
        pltpu.SemaphoreType.DMA,
    ],
)
def _emb_pe_kernel(x_hbm, table_hbm, pe_hbm, out_hbm, idx_v, pe_v, row_v, sem):
    wid = lax.axis_index("s") * NUM_CORES + lax.axis_index("c")
    pos0 = wid * POS_PER_WORKER

    def chunk_body(ci, carry):
        base = pos0 + ci * CHUNK
        pltpu.sync_copy(pe_hbm.at[pl.ds(base, CHUNK)], pe_v)
        for b in range(BATCH):
            pltpu.sync_copy(x_hbm.at[b, pl.ds(base, CHUNK)], idx_v)
            pltpu.async_copy(table_hbm.at[idx_v], row_v, sem).wait()

            def row_body(r, carry2):
                for g in range(D_GROUPS):
                    o = g * LANES
                    e = row_v[r, pl.ds(o, LANES)]
                    p = pe_v[r, pl.ds(o, LANES)]
                    row_v[r, pl.ds(o, LANES)] = e * SCALE + p
                return carry2

            lax.fori_loop(0, CHUNK, row_body, 0)
            pltpu.sync_copy(row_v, out_hbm.at[b, pl.ds(base, CHUNK)])
        return carry

    lax.fori_loop(0, N_CHUNKS, chunk_body, 0)


def kernel(x, table):
    pe = jnp.asarray(_PE)
    return _emb_pe_kernel(x.astype(jnp.int32), table, pe)


# speedup vs baseline: 1.3877x; 1.3877x over previous
import functools

import numpy as np
import jax
import jax.numpy as jnp
from jax import lax
from jax.experimental import pallas as pl
from jax.experimental.pallas import tpu as pltpu
from jax.experimental.pallas import tpu_sc as plsc

D_MODEL = 768
VOCAB = 100000
BATCH = 4
SEQ = 8192

SCALE = float(np.sqrt(np.float32(D_MODEL)))

NUM_CORES = 2
NUM_SUBCORES = 16
NUM_WORKERS = NUM_CORES * NUM_SUBCORES
POS_PER_WORKER = SEQ // NUM_WORKERS
CHUNK = 32
N_CHUNKS = POS_PER_WORKER // CHUNK
LANES = 16
D_GROUPS = D_MODEL // LANES


def _sinusoidal_pe(length, d_model):
    pos = np.arange(length)[:, None].astype(np.float32)
    i = np.arange(d_model)[None, :].astype(np.float32)
    angle_rates = 1.0 / np.power(10000.0, (2.0 * (i // 2)) / np.float32(d_model))
    angles = pos * angle_rates
    pe = np.zeros((length, d_model), dtype=np.float32)
    pe[:, 0::2] = np.sin(angles[:, 0::2])
    pe[:, 1::2] = np.cos(angles[:, 1::2])
    return pe


_PE = _sinusoidal_pe(SEQ, D_MODEL)

_MESH = plsc.VectorSubcoreMesh(core_axis_name="c", subcore_axis_name="s")


@functools.partial(
    pl.kernel,
    out_type=jax.ShapeDtypeStruct((BATCH, SEQ, D_MODEL), jnp.float32),
    mesh=_MESH,
    scratch_types=[
        pltpu.VMEM((BATCH, POS_PER_WORKER), jnp.int32),
        pltpu.VMEM((CHUNK, D_MODEL), jnp.float32),
        pltpu.VMEM((CHUNK, D_MODEL), jnp.float32),
        pltpu.VMEM((CHUNK, D_MODEL), jnp.float32),
        pltpu.VMEM((CHUNK, D_MODEL), jnp.float32),
        pltpu.SemaphoreType.DMA,
        pltpu.SemaphoreType.DMA,
        pltpu.SemaphoreType.DMA,
        pltpu.SemaphoreType.DMA,
        pltpu.SemaphoreType.DMA,
        pltpu.SemaphoreType.DMA,
    ],
)
def _emb_pe_kernel(x_hbm, table_hbm, pe_hbm, out_hbm,
                   idx_v, g0, g1, o0, o1,
                   gsem0, gsem1, fsem0, fsem1, ssem0, ssem1):
    gbuf = (g0, g1)
    obuf = (o0, o1)
    gsem = (gsem0, gsem1)
    fsem = (fsem0, fsem1)
    ssem = (ssem0, ssem1)

    wid = lax.axis_index("s") * NUM_CORES + lax.axis_index("c")
    pos0 = wid * POS_PER_WORKER

    def pe_src(ci):
        return pe_hbm.at[pl.ds(pos0 + ci * CHUNK, CHUNK)]

    def gather_src(ci, b):
        return table_hbm.at[idx_v.at[b, pl.ds(ci * CHUNK, CHUNK)]]

    def out_dst(ci, b):
        return out_hbm.at[b, pl.ds(pos0 + ci * CHUNK, CHUNK)]

    def issue_f(ci, b):
        pltpu.make_async_copy(pe_src(ci), obuf[b % 2], fsem[b % 2]).start()

    def wait_f(ci, b):
        pltpu.make_async_copy(pe_src(ci), obuf[b % 2], fsem[b % 2]).wait()

    def issue_g(ci, b):
        pltpu.make_async_copy(gather_src(ci, b), gbuf[b % 2], gsem[b % 2]).start()

    def wait_g(ci, b):
        pltpu.make_async_copy(gather_src(ci, b), gbuf[b % 2], gsem[b % 2]).wait()

    def issue_s(ci, b):
        pltpu.make_async_copy(obuf[b % 2], out_dst(ci, b), ssem[b % 2]).start()

    def wait_s(ci, b):
        pltpu.make_async_copy(obuf[b % 2], out_dst(ci, b), ssem[b % 2]).wait()

    def compute(b):
        g = gbuf[b % 2]
        o = obuf[b % 2]

        def row_body(r, carry):
            for gi in range(D_GROUPS):
                sl = pl.ds(gi * LANES, LANES)
                plsc.addupdate(o.at[r, sl], g[r, sl] * SCALE)
            return carry

        lax.fori_loop(0, CHUNK, row_body, 0)

    def nxt(ci, b, k):
        return (ci + (b + k) // BATCH, (b + k) % BATCH)

    def step(ci, b, first=False, f_next=True, g_next2=True):
        if not first:
            pci, pb = nxt(ci, b, -1) if b > 0 else (ci - 1, BATCH - 1)
            wait_s(pci, pb)
        if f_next:
            fci, fb = nxt(ci, b, 1)
            issue_f(fci, fb)
        wait_g(ci, b)
        wait_f(ci, b)
        compute(b)
        issue_s(ci, b)
        if g_next2:
            gci, gb = nxt(ci, b, 2)
            issue_g(gci, gb)

    for b in range(BATCH):
        pltpu.sync_copy(x_hbm.at[b, pl.ds(pos0, POS_PER_WORKER)],
                        idx_v.at[b])

    issue_g(0, 0)
    issue_g(0, 1)
    issue_f(0, 0)

    step(0, 0, first=True)
    step(0, 1)
    step(0, 2)
    step(0, 3)

    def chunk_body(ci, carry):
        step(ci, 0)
        step(ci, 1)
        step(ci, 2)
        step(ci, 3)
        return carry

    lax.fori_loop(1, N_CHUNKS - 1, chunk_body, 0)

    lc = N_CHUNKS - 1
    step(lc, 0)
    step(lc, 1)
    step(lc, 2, g_next2=False)
    step(lc, 3, f_next=False, g_next2=False)

    wait_s(lc, 3)


def kernel(x, table):
    pe = jnp.asarray(_PE)
    return _emb_pe_kernel(x.astype(jnp.int32), table, pe)
